# baseline (speedup 1.0000x reference)
import functools
import jax
import jax.numpy as jnp
from jax import lax
from jax.experimental import pallas as pl
from jax.experimental.pallas import tpu as pltpu
from jax.experimental.pallas import tpu_sc as plsc

_K = 8
_NBUF = 2


def kernel(input):
    B, C, H, W = input.shape
    NC, NS = 2, 16
    half = C // 2
    G = half // _K

    mesh = plsc.VectorSubcoreMesh(core_axis_name="c", subcore_axis_name="s")

    @functools.partial(
        pl.kernel,
        out_type=jax.ShapeDtypeStruct((B, C, H, W), jnp.float32),
        mesh=mesh,
        scratch_types=[
            pltpu.VMEM((_NBUF, _K, H, W), jnp.float32),
            pltpu.SemaphoreType.DMA((_NBUF,)),
            pltpu.SemaphoreType.DMA((_NBUF,)),
        ],
        compiler_params=pltpu.CompilerParams(use_tc_tiling_on_sc=True),
    )
    def _sc_reverse(x_hbm, o_hbm, bufs, rsem, wsem):
        wid = lax.axis_index("s") * NC + lax.axis_index("c")
        b = wid // 2
        c0 = (wid % 2) * half
        rb0 = C - c0 - half

        def fire_read(g):
            slot = lax.rem(g, _NBUF)
            rbase = rb0 + g * _K
            pltpu.async_copy(
                x_hbm.at[b, pl.ds(rbase, _K)], bufs.at[slot], rsem.at[slot]
            )

        def wait_read(g):
            slot = lax.rem(g, _NBUF)
            pltpu.make_async_copy(
                x_hbm.at[b, pl.ds(0, _K)], bufs.at[slot], rsem.at[slot]
            ).wait()

        def fire_writes(g):
            slot = lax.rem(g, _NBUF)
            rbase = rb0 + g * _K
            for k in range(_K):
                cout = C - 1 - (rbase + k)
                pltpu.async_copy(
                    bufs.at[slot, k], o_hbm.at[b, cout], wsem.at[slot]
                )

        def wait_writes(g):
            slot = lax.rem(g, _NBUF)
            for _ in range(_K):
                pltpu.make_async_copy(
                    bufs.at[slot, 0], o_hbm.at[b, 0], wsem.at[slot]
                ).wait()

        fire_read(0)

        def body(g, carry):
            nxt = g + 1

            @pl.when(nxt < G)
            def _():
                @pl.when(nxt >= _NBUF)
                def _():
                    wait_writes(nxt - _NBUF)

                fire_read(nxt)

            wait_read(g)
            fire_writes(g)
            return carry

        lax.fori_loop(0, G, body, 0)
        for j in range(_NBUF):
            wait_writes(G - _NBUF + j)

    return _sc_reverse(input)


# speedup vs baseline: 3.9719x; 3.9719x over previous
import jax
import jax.numpy as jnp
from jax import lax
from jax.experimental import pallas as pl

_ROWS = 448
_LG = 128
_NG = 6


def _rev_body(x_ref, p_ref, o_ref):
    p = p_ref[...]
    for g in range(_NG):
        o_ref[:, (_NG - 1 - g) * _LG:(_NG - g) * _LG] = lax.dot(
            x_ref[:, g * _LG:(g + 1) * _LG],
            p,
            precision=lax.Precision.HIGHEST,
            preferred_element_type=jnp.float32,
        )


def kernel(input):
    B, C, H, W = input.shape
    x2 = input.transpose(0, 2, 3, 1).reshape(B * H * W, C)
    nblk = (B * H * W) // _ROWS
    p = jnp.flip(jnp.eye(_LG, dtype=jnp.float32), axis=1)
    out2 = pl.pallas_call(
        _rev_body,
        grid=(nblk,),
        in_specs=[
            pl.BlockSpec((_ROWS, C), lambda i: (i, 0)),
            pl.BlockSpec((_LG, _LG), lambda i: (0, 0)),
        ],
        out_specs=pl.BlockSpec((_ROWS, C), lambda i: (i, 0)),
        out_shape=jax.ShapeDtypeStruct((B * H * W, C), jnp.float32),
    )(x2, p)
    return out2.reshape(B, H, W, C).transpose(0, 3, 1, 2)


# speedup vs baseline: 5.2558x; 1.3232x over previous
import jax
import jax.numpy as jnp
from jax import lax
from jax.experimental import pallas as pl

_ROWS = 1792
_LG = 128
_NG = 6


def _rev_body(x_ref, p_ref, o_ref):
    p = p_ref[...]
    for g in range(_NG):
        o_ref[:, (_NG - 1 - g) * _LG:(_NG - g) * _LG] = lax.dot(
            x_ref[:, g * _LG:(g + 1) * _LG],
            p,
            precision=lax.Precision.HIGHEST,
            preferred_element_type=jnp.float32,
        )


def kernel(input):
    B, C, H, W = input.shape
    x2 = input.transpose(0, 2, 3, 1).reshape(B * H * W, C)
    nblk = (B * H * W) // _ROWS
    p = jnp.flip(jnp.eye(_LG, dtype=jnp.float32), axis=1)
    out2 = pl.pallas_call(
        _rev_body,
        grid=(nblk,),
        in_specs=[
            pl.BlockSpec((_ROWS, C), lambda i: (i, 0)),
            pl.BlockSpec((_LG, _LG), lambda i: (0, 0)),
        ],
        out_specs=pl.BlockSpec((_ROWS, C), lambda i: (i, 0)),
        out_shape=jax.ShapeDtypeStruct((B * H * W, C), jnp.float32),
    )(x2, p)
    return out2.reshape(B, H, W, C).transpose(0, 3, 1, 2)
